# baseline (speedup 1.0000x reference)
import functools

import jax
import jax.numpy as jnp
from jax import lax
from jax.experimental import pallas as pl
from jax.experimental.pallas import tpu as pltpu
import jax.experimental.pallas.tpu_sc as plsc

_N = 10000
_NP = 10240
_E = 320000
_D = 128
_CHUNK = 128
_NW = 32
_EPT = 10240
_EPAD = _EPT * _NW
_NCHUNKS = _EPT // _CHUNK
_NPAIR = _NCHUNKS // 2
_EPT_H = 10112
_EPAD_H = _EPT_H * _NW
_NCHUNKS_H = _EPT_H // _CHUNK
_RPS = _NP // 16
_BLK = 1024
_GRID = _NP // _BLK

def _make_mesh():
    return plsc.VectorSubcoreMesh(core_axis_name="c", subcore_axis_name="s",
                                  num_cores=2, num_subcores=16)


@functools.cache
def _get_sc_hist():
    @functools.partial(
        pl.kernel,
        out_type=(
            jax.ShapeDtypeStruct((2, _NP, 16), jnp.float32),
            jax.ShapeDtypeStruct((2, _NP, 16), jnp.float32),
        ),
        mesh=_make_mesh(),
        scratch_types=[
            pltpu.VMEM((_CHUNK,), jnp.int32),
            pltpu.VMEM((_CHUNK,), jnp.int32),
            pltpu.VMEM((_CHUNK, 16), jnp.float32),
            pltpu.VMEM((_CHUNK, 16), jnp.float32),
            pltpu.VMEM_SHARED((_NP, 16), jnp.float32),
            pltpu.VMEM_SHARED((_NP, 16), jnp.float32),
        ],
    )
    def _sc_hist(src_h, dst_h, deg_out, cnt_out, dv, sv, ones_b, zero_b,
                 acc_deg, acc_cnt):
        c = lax.axis_index("c")
        s = lax.axis_index("s")
        w = s * 2 + c

        one16 = jnp.ones((16,), jnp.float32)
        zero16 = jnp.zeros((16,), jnp.float32)

        def fill(i, _):
            ones_b[i, pl.ds(0, 16)] = one16
            zero_b[i, pl.ds(0, 16)] = zero16
            return 0

        lax.fori_loop(0, _CHUNK, fill, 0)
        for t in range(_RPS // _CHUNK):
            pltpu.sync_copy(zero_b,
                            acc_deg.at[pl.ds(s * _RPS + t * _CHUNK, _CHUNK)])
            pltpu.sync_copy(zero_b,
                            acc_cnt.at[pl.ds(s * _RPS + t * _CHUNK, _CHUNK)])
        plsc.subcore_barrier()

        def chunk(i, _):
            base = pl.multiple_of(w * _EPT_H + i * _CHUNK, _CHUNK)
            pltpu.sync_copy(dst_h.at[pl.ds(base, _CHUNK)], dv)
            pltpu.sync_copy(src_h.at[pl.ds(base, _CHUNK)], sv)
            pltpu.sync_copy(ones_b, acc_deg.at[dv], add=True)
            pltpu.sync_copy(ones_b, acc_cnt.at[sv], add=True)
            return 0

        lax.fori_loop(0, _NCHUNKS_H, chunk, 0)
        plsc.subcore_barrier()
        pltpu.sync_copy(acc_deg.at[pl.ds(s * _RPS, _RPS)],
                        deg_out.at[c, pl.ds(s * _RPS, _RPS)])
        pltpu.sync_copy(acc_cnt.at[pl.ds(s * _RPS, _RPS)],
                        cnt_out.at[c, pl.ds(s * _RPS, _RPS)])

    return _sc_hist


@functools.cache
def _get_sc_edge_sum():
    @functools.partial(
        pl.kernel,
        out_type=jax.ShapeDtypeStruct((2, _NP, _D), jnp.float32),
        mesh=_make_mesh(),
        scratch_types=[
            pltpu.VMEM((_CHUNK,), jnp.int32),
            pltpu.VMEM((_CHUNK,), jnp.int32),
            pltpu.VMEM((_CHUNK,), jnp.int32),
            pltpu.VMEM((_CHUNK,), jnp.int32),
            pltpu.VMEM((_CHUNK, _D), jnp.float32),
            pltpu.VMEM((_CHUNK, _D), jnp.float32),
            pltpu.VMEM_SHARED((_NP, _D), jnp.float32),
            pltpu.SemaphoreType.DMA,
            pltpu.SemaphoreType.DMA,
            pltpu.SemaphoreType.DMA,
        ],
    )
    def _sc_edge_sum(table_h, gidx_h, sidx_h, out, gv_a, gv_b, sv_a, sv_b,
                     rows_a, rows_b, accum, sem_a, sem_b, sem_i):
        c = lax.axis_index("c")
        s = lax.axis_index("s")
        w = s * 2 + c

        zero16 = jnp.zeros((16,), jnp.float32)

        def fill(i, _):
            for j in range(_D // 16):
                rows_a[i, pl.ds(j * 16, 16)] = zero16
            return 0

        lax.fori_loop(0, _CHUNK, fill, 0)
        for t in range(_RPS // _CHUNK):
            pltpu.sync_copy(rows_a,
                            accum.at[pl.ds(s * _RPS + t * _CHUNK, _CHUNK)])
        pltpu.sync_copy(gidx_h.at[w, 0], gv_a)
        pltpu.sync_copy(sidx_h.at[w, 0], sv_a)
        plsc.subcore_barrier()

        def pair(i, _):
            a = 2 * i
            b = 2 * i + 1
            da = pltpu.async_copy(table_h.at[gv_a], rows_a, sem_a)
            i1 = pltpu.async_copy(gidx_h.at[w, b], gv_b, sem_i)
            i2 = pltpu.async_copy(sidx_h.at[w, b], sv_b, sem_i)
            da.wait()
            i1.wait()
            i2.wait()
            db = pltpu.async_copy(table_h.at[gv_b], rows_b, sem_b)
            pltpu.sync_copy(rows_a, accum.at[sv_a], add=True)
            i3 = pltpu.async_copy(gidx_h.at[w, a + 2], gv_a, sem_i)
            i4 = pltpu.async_copy(sidx_h.at[w, a + 2], sv_a, sem_i)
            db.wait()
            i3.wait()
            i4.wait()
            pltpu.sync_copy(rows_b, accum.at[sv_b], add=True)
            return 0

        lax.fori_loop(0, _NPAIR, pair, 0)
        plsc.subcore_barrier()
        pltpu.sync_copy(accum.at[pl.ds(s * _RPS, _RPS)],
                        out.at[c, pl.ds(s * _RPS, _RPS)])

    return _sc_edge_sum


def _dot(a, b):
    return lax.dot_general(a, b, (((1,), (0,)), ((), ())),
                           preferred_element_type=jnp.float32,
                           precision=lax.Precision.HIGHEST)


def _dinv_from(dp):
    deg = dp[0][:, 0:1] + dp[1][:, 0:1] + 1.0
    return 1.0 / jnp.sqrt(deg)


def _cinv_from(cp):
    cnt = cp[0][:, 0:1] + cp[1][:, 0:1]
    return 1.0 / jnp.maximum(cnt, 1.0)


def _row_spec(width=_D):
    return pl.BlockSpec((_BLK, width), lambda i: (i, 0))


def _parts_spec(width=_D):
    return pl.BlockSpec((2, _BLK, width), lambda i: (0, i, 0))


def _full_spec(shape):
    return pl.BlockSpec(shape, lambda i: tuple(0 for _ in shape))


def _tc_pre(xp, W_res, W0, dparts):
    def body(x_ref, wr_ref, w0_ref, dp_ref, x0_ref, y0p_ref):
        x = x_ref[...]
        x0_ref[...] = jnp.maximum(_dot(x, wr_ref[...]), 0.0)
        y0p_ref[...] = _dinv_from(dp_ref) * _dot(x, w0_ref[...])

    return pl.pallas_call(
        body,
        grid=(_GRID,),
        in_specs=[_row_spec(), _full_spec((_D, _D)), _full_spec((_D, _D)),
                  _parts_spec(16)],
        out_specs=[_row_spec(), _row_spec()],
        out_shape=[jax.ShapeDtypeStruct((_NP, _D), jnp.float32),
                   jax.ShapeDtypeStruct((_NP, _D), jnp.float32)],
    )(xp, W_res, W0, dparts)


def _tc_conv_out(parts, yp, dparts, b, relu):
    def body(p_ref, y_ref, dp_ref, b_ref, o_ref):
        h = _dinv_from(dp_ref) * (p_ref[0] + p_ref[1] + y_ref[...]) + b_ref[...]
        o_ref[...] = jnp.maximum(h, 0.0) if relu else h

    return pl.pallas_call(
        body,
        grid=(_GRID,),
        in_specs=[_parts_spec(), _row_spec(), _parts_spec(16),
                  _full_spec((1, _D))],
        out_specs=_row_spec(),
        out_shape=jax.ShapeDtypeStruct((_NP, _D), jnp.float32),
    )(parts, yp, dparts, b)


def _tc_mean_dev(mparts, cparts, h):
    def body(m_ref, cp_ref, h_ref, xm_ref, z_ref):
        xm = (m_ref[0] + m_ref[1]) * _cinv_from(cp_ref)
        xm_ref[...] = xm
        z_ref[...] = jnp.abs(h_ref[...] - xm)

    return pl.pallas_call(
        body,
        grid=(_GRID,),
        in_specs=[_parts_spec(), _parts_spec(16), _row_spec()],
        out_specs=[_row_spec(), _row_spec()],
        out_shape=[jax.ShapeDtypeStruct((_NP, _D), jnp.float32),
                   jax.ShapeDtypeStruct((_NP, _D), jnp.float32)],
    )(mparts, cparts, h)


def _tc_gate(sparts, cparts, xm, h, x0, wsc, Wn, dparts):
    def body(sp_ref, cp_ref, xm_ref, h_ref, x0_ref, ws_ref, wn_ref, dp_ref,
             xcat_ref, ynp_ref):
        h = h_ref[...]
        xm = xm_ref[...]
        xs = (sp_ref[0] + sp_ref[1]) * _cinv_from(cp_ref)
        t = jnp.sum(xm * h * ws_ref[0:1, :] + xs * ws_ref[1:2, :]
                    + h * ws_ref[2:3, :], axis=-1, keepdims=True)
        sig = 1.0 / (1.0 + jnp.exp(-t))
        a = (1.0 - sig) * h
        bb = sig * x0_ref[...]
        nrm = jnp.sqrt(jnp.sum(a * a, axis=-1, keepdims=True)
                       + jnp.sum(bb * bb, axis=-1, keepdims=True))
        r = 1.0 / jnp.maximum(nrm, 1e-12)
        a = a * r
        bb = bb * r
        xcat_ref[:, 0:_D] = a
        xcat_ref[:, _D:2 * _D] = bb
        ynp_ref[...] = _dinv_from(dp_ref) * (
            _dot(a, wn_ref[0:_D, :]) + _dot(bb, wn_ref[_D:2 * _D, :]))

    return pl.pallas_call(
        body,
        grid=(_GRID,),
        in_specs=[_parts_spec(), _parts_spec(16), _row_spec(), _row_spec(),
                  _row_spec(), _full_spec((8, _D)), _full_spec((2 * _D, _D)),
                  _parts_spec(16)],
        out_specs=[_row_spec(2 * _D), _row_spec()],
        out_shape=[jax.ShapeDtypeStruct((_NP, 2 * _D), jnp.float32),
                   jax.ShapeDtypeStruct((_NP, _D), jnp.float32)],
    )(sparts, cparts, xm, h, x0, wsc, Wn, dparts)


def kernel(x, edge_index, W_res, W0, b0, W1, b1, W2, b2, W_ws):
    src = edge_index[0]
    dst = edge_index[1]
    pad = jnp.full((_EPAD - _E,), _N, dtype=jnp.int32)
    srcp = jnp.concatenate([src, pad]).reshape(_NW, _NCHUNKS, _CHUNK)
    dstp = jnp.concatenate([dst, pad]).reshape(_NW, _NCHUNKS, _CHUNK)
    extra = jnp.full((_NW, 1, _CHUNK), _N, dtype=jnp.int32)
    src_g = jnp.concatenate([srcp, extra], axis=1)
    dst_g = jnp.concatenate([dstp, extra], axis=1)
    pad_h = jnp.full((_EPAD_H - _E,), _N, dtype=jnp.int32)
    src_f = jnp.concatenate([src, pad_h])
    dst_f = jnp.concatenate([dst, pad_h])
    xp = jnp.pad(x, ((0, _NP - _N), (0, 0)))
    b0r = b0.reshape(1, _D)
    b1r = b1.reshape(1, _D)
    b2r = b2.reshape(1, _D)
    wsc = jnp.pad(W_ws.reshape(3, _D), ((0, 5), (0, 0)))

    dparts, cparts = _get_sc_hist()(src_f, dst_f)
    x0, y0p = _tc_pre(xp, W_res, W0, dparts)

    p = _get_sc_edge_sum()(y0p, src_g, dst_g)
    h0 = _tc_conv_out(p, y0p, dparts, b0r, relu=True)
    m = _get_sc_edge_sum()(h0, dst_g, src_g)
    xm, z = _tc_mean_dev(m, cparts, h0)
    sp = _get_sc_edge_sum()(z, dst_g, src_g)
    x1, y1p = _tc_gate(sp, cparts, xm, h0, x0, wsc, W1, dparts)

    p = _get_sc_edge_sum()(y1p, src_g, dst_g)
    h1 = _tc_conv_out(p, y1p, dparts, b1r, relu=True)
    m = _get_sc_edge_sum()(h1, dst_g, src_g)
    xm, z = _tc_mean_dev(m, cparts, h1)
    sp = _get_sc_edge_sum()(z, dst_g, src_g)
    x2, y2p = _tc_gate(sp, cparts, xm, h1, x0, wsc, W2, dparts)

    p = _get_sc_edge_sum()(y2p, src_g, dst_g)
    out = _tc_conv_out(p, y2p, dparts, b2r, relu=False)

    return (out[:_N], h0[:_N], x1[:_N], h1[:_N], x2[:_N])


# speedup vs baseline: 2.8418x; 2.8418x over previous
import functools

import jax
import jax.numpy as jnp
from jax import lax
from jax.experimental import pallas as pl
from jax.experimental.pallas import tpu as pltpu
import jax.experimental.pallas.tpu_sc as plsc

_N = 10000
_NP = 10240
_E = 320000
_D = 128
_CHUNK = 128
_NW = 32
_EPT = 10240
_EPAD = _EPT * _NW
_NCHUNKS = _EPT // _CHUNK
_NPAIR = _NCHUNKS // 2
_EPT_H = 10112
_EPAD_H = _EPT_H * _NW
_NCHUNKS_H = _EPT_H // _CHUNK
_RPS = _NP // 16
_BLK = 1024
_GRID = _NP // _BLK

def _make_mesh():
    return plsc.VectorSubcoreMesh(core_axis_name="c", subcore_axis_name="s",
                                  num_cores=2, num_subcores=16)


@functools.cache
def _get_sc_hist():
    @functools.partial(
        pl.kernel,
        out_type=(
            jax.ShapeDtypeStruct((2, _NP, 16), jnp.float32),
            jax.ShapeDtypeStruct((2, _NP, 16), jnp.float32),
        ),
        mesh=_make_mesh(),
        scratch_types=[
            pltpu.VMEM((_CHUNK,), jnp.int32),
            pltpu.VMEM((_CHUNK,), jnp.int32),
            pltpu.VMEM((_CHUNK, 16), jnp.float32),
            pltpu.VMEM((_CHUNK, 16), jnp.float32),
            pltpu.VMEM_SHARED((_NP, 16), jnp.float32),
            pltpu.VMEM_SHARED((_NP, 16), jnp.float32),
        ],
    )
    def _sc_hist(src_h, dst_h, deg_out, cnt_out, dv, sv, ones_b, zero_b,
                 acc_deg, acc_cnt):
        c = lax.axis_index("c")
        s = lax.axis_index("s")
        w = s * 2 + c

        one16 = jnp.ones((16,), jnp.float32)
        zero16 = jnp.zeros((16,), jnp.float32)

        def fill(i, _):
            ones_b[i, pl.ds(0, 16)] = one16
            zero_b[i, pl.ds(0, 16)] = zero16
            return 0

        lax.fori_loop(0, _CHUNK, fill, 0)
        for t in range(_RPS // _CHUNK):
            pltpu.sync_copy(zero_b,
                            acc_deg.at[pl.ds(s * _RPS + t * _CHUNK, _CHUNK)])
            pltpu.sync_copy(zero_b,
                            acc_cnt.at[pl.ds(s * _RPS + t * _CHUNK, _CHUNK)])
        plsc.subcore_barrier()

        def chunk(i, _):
            base = pl.multiple_of(w * _EPT_H + i * _CHUNK, _CHUNK)
            pltpu.sync_copy(dst_h.at[pl.ds(base, _CHUNK)], dv)
            pltpu.sync_copy(src_h.at[pl.ds(base, _CHUNK)], sv)
            pltpu.sync_copy(ones_b, acc_deg.at[dv], add=True)
            pltpu.sync_copy(ones_b, acc_cnt.at[sv], add=True)
            return 0

        lax.fori_loop(0, _NCHUNKS_H, chunk, 0)
        plsc.subcore_barrier()
        pltpu.sync_copy(acc_deg.at[pl.ds(s * _RPS, _RPS)],
                        deg_out.at[c, pl.ds(s * _RPS, _RPS)])
        pltpu.sync_copy(acc_cnt.at[pl.ds(s * _RPS, _RPS)],
                        cnt_out.at[c, pl.ds(s * _RPS, _RPS)])

    return _sc_hist


@functools.cache
def _get_sc_edge_sum():
    @functools.partial(
        pl.kernel,
        out_type=jax.ShapeDtypeStruct((2, _NP, _D), jnp.float32),
        mesh=_make_mesh(),
        scratch_types=[
            pltpu.VMEM((_CHUNK,), jnp.int32),
            pltpu.VMEM((_CHUNK,), jnp.int32),
            pltpu.VMEM((_CHUNK,), jnp.int32),
            pltpu.VMEM((_CHUNK,), jnp.int32),
            pltpu.VMEM((_CHUNK, _D), jnp.float32),
            pltpu.VMEM((_CHUNK, _D), jnp.float32),
            pltpu.VMEM_SHARED((_NP, _D), jnp.float32),
            pltpu.SemaphoreType.DMA,
            pltpu.SemaphoreType.DMA,
            pltpu.SemaphoreType.DMA,
        ],
    )
    def _sc_edge_sum(table_h, gidx_h, sidx_h, out, gv_a, gv_b, sv_a, sv_b,
                     rows_a, rows_b, accum, sem_a, sem_b, sem_i):
        c = lax.axis_index("c")
        s = lax.axis_index("s")
        w = s * 2 + c

        zero16 = jnp.zeros((16,), jnp.float32)

        def fill(i, _):
            for j in range(_D // 16):
                rows_a[i, pl.ds(j * 16, 16)] = zero16
            return 0

        lax.fori_loop(0, _CHUNK, fill, 0)
        for t in range(_RPS // _CHUNK):
            pltpu.sync_copy(rows_a,
                            accum.at[pl.ds(s * _RPS + t * _CHUNK, _CHUNK)])
        pltpu.sync_copy(gidx_h.at[w, 0], gv_a)
        pltpu.sync_copy(sidx_h.at[w, 0], sv_a)
        plsc.subcore_barrier()

        def pair(i, _):
            a = 2 * i
            b = 2 * i + 1
            da = pltpu.async_copy(table_h.at[gv_a], rows_a, sem_a)
            i1 = pltpu.async_copy(gidx_h.at[w, b], gv_b, sem_i)
            i2 = pltpu.async_copy(sidx_h.at[w, b], sv_b, sem_i)
            da.wait()
            i1.wait()
            i2.wait()
            db = pltpu.async_copy(table_h.at[gv_b], rows_b, sem_b)
            pltpu.sync_copy(rows_a, accum.at[sv_a], add=True)
            i3 = pltpu.async_copy(gidx_h.at[w, a + 2], gv_a, sem_i)
            i4 = pltpu.async_copy(sidx_h.at[w, a + 2], sv_a, sem_i)
            db.wait()
            i3.wait()
            i4.wait()
            pltpu.sync_copy(rows_b, accum.at[sv_b], add=True)
            return 0

        lax.fori_loop(0, _NPAIR, pair, 0)
        plsc.subcore_barrier()
        pltpu.sync_copy(accum.at[pl.ds(s * _RPS, _RPS)],
                        out.at[c, pl.ds(s * _RPS, _RPS)])

    return _sc_edge_sum


def _dot(a, b):
    return lax.dot_general(a, b, (((1,), (0,)), ((), ())),
                           preferred_element_type=jnp.float32,
                           precision=lax.Precision.HIGHEST)


def _dinv_from(dp):
    deg = dp[0][:, 0:1] + dp[1][:, 0:1] + 1.0
    return 1.0 / jnp.sqrt(deg)


def _cinv_from(cp):
    cnt = cp[0][:, 0:1] + cp[1][:, 0:1]
    return 1.0 / jnp.maximum(cnt, 1.0)


def _row_spec(width=_D):
    return pl.BlockSpec((_BLK, width), lambda i: (i, 0))


def _parts_spec(width=_D):
    return pl.BlockSpec((2, _BLK, width), lambda i: (0, i, 0))


def _full_spec(shape):
    return pl.BlockSpec(shape, lambda i: tuple(0 for _ in shape))


def _tc_pre(xp, W_res, W0, dparts):
    def body(x_ref, wr_ref, w0_ref, dp_ref, x0_ref, y0p_ref):
        x = x_ref[...]
        x0_ref[...] = jnp.maximum(_dot(x, wr_ref[...]), 0.0)
        y0p_ref[...] = _dinv_from(dp_ref) * _dot(x, w0_ref[...])

    return pl.pallas_call(
        body,
        grid=(_GRID,),
        in_specs=[_row_spec(), _full_spec((_D, _D)), _full_spec((_D, _D)),
                  _parts_spec(16)],
        out_specs=[_row_spec(), _row_spec()],
        out_shape=[jax.ShapeDtypeStruct((_NP, _D), jnp.float32),
                   jax.ShapeDtypeStruct((_NP, _D), jnp.float32)],
    )(xp, W_res, W0, dparts)


def _tc_conv_out(parts, yp, dparts, b, relu):
    def body(p_ref, y_ref, dp_ref, b_ref, o_ref):
        h = _dinv_from(dp_ref) * (p_ref[0] + p_ref[1] + y_ref[...]) + b_ref[...]
        o_ref[...] = jnp.maximum(h, 0.0) if relu else h

    return pl.pallas_call(
        body,
        grid=(_GRID,),
        in_specs=[_parts_spec(), _row_spec(), _parts_spec(16),
                  _full_spec((1, _D))],
        out_specs=_row_spec(),
        out_shape=jax.ShapeDtypeStruct((_NP, _D), jnp.float32),
    )(parts, yp, dparts, b)


def _tc_mean_dev(mparts, cparts, h):
    def body(m_ref, cp_ref, h_ref, xm_ref, z_ref):
        xm = (m_ref[0] + m_ref[1]) * _cinv_from(cp_ref)
        xm_ref[...] = xm
        z_ref[...] = jnp.abs(h_ref[...] - xm)

    return pl.pallas_call(
        body,
        grid=(_GRID,),
        in_specs=[_parts_spec(), _parts_spec(16), _row_spec()],
        out_specs=[_row_spec(), _row_spec()],
        out_shape=[jax.ShapeDtypeStruct((_NP, _D), jnp.float32),
                   jax.ShapeDtypeStruct((_NP, _D), jnp.float32)],
    )(mparts, cparts, h)


def _tc_gate(sparts, cparts, xm, h, x0, wsc, Wn, dparts):
    def body(sp_ref, cp_ref, xm_ref, h_ref, x0_ref, ws_ref, wn_ref, dp_ref,
             xcat_ref, ynp_ref):
        h = h_ref[...]
        xm = xm_ref[...]
        xs = (sp_ref[0] + sp_ref[1]) * _cinv_from(cp_ref)
        t = jnp.sum(xm * h * ws_ref[0:1, :] + xs * ws_ref[1:2, :]
                    + h * ws_ref[2:3, :], axis=-1, keepdims=True)
        sig = 1.0 / (1.0 + jnp.exp(-t))
        a = (1.0 - sig) * h
        bb = sig * x0_ref[...]
        nrm = jnp.sqrt(jnp.sum(a * a, axis=-1, keepdims=True)
                       + jnp.sum(bb * bb, axis=-1, keepdims=True))
        r = 1.0 / jnp.maximum(nrm, 1e-12)
        a = a * r
        bb = bb * r
        xcat_ref[:, 0:_D] = a
        xcat_ref[:, _D:2 * _D] = bb
        ynp_ref[...] = _dinv_from(dp_ref) * (
            _dot(a, wn_ref[0:_D, :]) + _dot(bb, wn_ref[_D:2 * _D, :]))

    return pl.pallas_call(
        body,
        grid=(_GRID,),
        in_specs=[_parts_spec(), _parts_spec(16), _row_spec(), _row_spec(),
                  _row_spec(), _full_spec((8, _D)), _full_spec((2 * _D, _D)),
                  _parts_spec(16)],
        out_specs=[_row_spec(2 * _D), _row_spec()],
        out_shape=[jax.ShapeDtypeStruct((_NP, 2 * _D), jnp.float32),
                   jax.ShapeDtypeStruct((_NP, _D), jnp.float32)],
    )(sparts, cparts, xm, h, x0, wsc, Wn, dparts)


def kernel(x, edge_index, W_res, W0, b0, W1, b1, W2, b2, W_ws):
    src = edge_index[0]
    dst = edge_index[1]
    pad = _N + (jnp.arange(_EPAD - _E, dtype=jnp.int32) % (_NP - _N))
    srcp = jnp.concatenate([src, pad]).reshape(_NW, _NCHUNKS, _CHUNK)
    dstp = jnp.concatenate([dst, pad]).reshape(_NW, _NCHUNKS, _CHUNK)
    extra = _N + (jnp.arange(_NW * _CHUNK, dtype=jnp.int32) % (_NP - _N))
    extra = extra.reshape(_NW, 1, _CHUNK)
    src_g = jnp.concatenate([srcp, extra], axis=1)
    dst_g = jnp.concatenate([dstp, extra], axis=1)
    pad_h = _N + (jnp.arange(_EPAD_H - _E, dtype=jnp.int32) % (_NP - _N))
    src_f = jnp.concatenate([src, pad_h])
    dst_f = jnp.concatenate([dst, pad_h])
    xp = jnp.pad(x, ((0, _NP - _N), (0, 0)))
    b0r = b0.reshape(1, _D)
    b1r = b1.reshape(1, _D)
    b2r = b2.reshape(1, _D)
    wsc = jnp.pad(W_ws.reshape(3, _D), ((0, 5), (0, 0)))

    dparts, cparts = _get_sc_hist()(src_f, dst_f)
    x0, y0p = _tc_pre(xp, W_res, W0, dparts)

    p = _get_sc_edge_sum()(y0p, src_g, dst_g)
    h0 = _tc_conv_out(p, y0p, dparts, b0r, relu=True)
    m = _get_sc_edge_sum()(h0, dst_g, src_g)
    xm, z = _tc_mean_dev(m, cparts, h0)
    sp = _get_sc_edge_sum()(z, dst_g, src_g)
    x1, y1p = _tc_gate(sp, cparts, xm, h0, x0, wsc, W1, dparts)

    p = _get_sc_edge_sum()(y1p, src_g, dst_g)
    h1 = _tc_conv_out(p, y1p, dparts, b1r, relu=True)
    m = _get_sc_edge_sum()(h1, dst_g, src_g)
    xm, z = _tc_mean_dev(m, cparts, h1)
    sp = _get_sc_edge_sum()(z, dst_g, src_g)
    x2, y2p = _tc_gate(sp, cparts, xm, h1, x0, wsc, W2, dparts)

    p = _get_sc_edge_sum()(y2p, src_g, dst_g)
    out = _tc_conv_out(p, y2p, dparts, b2r, relu=False)

    return (out[:_N], h0[:_N], x1[:_N], h1[:_N], x2[:_N])
